# baseline (speedup 1.0000x reference)
import functools

import jax
import jax.numpy as jnp
from jax import lax
from jax.experimental import pallas as pl
from jax.experimental.pallas import tpu as pltpu
from jax.experimental.pallas import tpu_sc as plsc

N = 100000
D = 128
S = 256
L = 16
NC = 2
NS = 16
NW = NC * NS
SEG_PER_W = S // NW
R = 64
NBUF = 8
KV = D // L
PIECE = 6248
SL = 6288
NT = 17


def _body(x_hbm, batch_hbm, wb_hbm, out_hbm, slice_v, lbv_v, lbtab_v, xbuf_v,
          wb_v, stage_v, bnd_s, lbsh_sh, sem):
    cid = lax.axis_index("c")
    sid = lax.axis_index("s")
    wid = cid * NS + sid

    pltpu.sync_copy(wb_hbm, wb_v)

    bstart = pl.multiple_of(jnp.minimum(sid * PIECE, N - SL), 8)
    pltpu.sync_copy(batch_hbm.at[pl.ds(bstart, SL)], slice_v)
    poff = sid * PIECE - bstart
    plen = jnp.where(sid == NS - 1, N - (NS - 1) * PIECE, PIECE)

    for g in range(NT):
        tg = g * L + lax.iota(jnp.int32, L)
        lo0 = jnp.full((L,), poff, jnp.int32)
        hi0 = jnp.full((L,), poff + plen, jnp.int32)

        def sbody(_, c):
            lo, hi = c
            act = lo < hi
            mid = lax.shift_right_logical(lo + hi, 1)
            vals = plsc.load_gather(slice_v, [jnp.minimum(mid, SL - 1)])
            less = vals < tg
            lo = jnp.where(act & less, mid + 1, lo)
            hi = jnp.where(act & (~less), mid, hi)
            return lo, hi

        lo, _ = lax.fori_loop(0, 13, sbody, (lo0, hi0))
        lbv_v[pl.ds(g * L, L)] = lo - poff

    pltpu.sync_copy(lbv_v, lbsh_sh.at[pl.ds(sid * NT * L, NT * L)])
    plsc.subcore_barrier()
    pltpu.sync_copy(lbsh_sh, lbtab_v)

    g0 = lax.shift_right_logical(wid, 1)
    par = wid & 1
    offa = pl.multiple_of(g0 * L, 16)
    offb = pl.multiple_of(jnp.minimum(g0 + 1, NT - 1) * L, 16)
    ba = lbtab_v[pl.ds(offa, L)]
    bb = lbtab_v[pl.ds(offb, L)]
    for t in range(1, NS):
        ba = ba + lbtab_v[pl.ds(t * NT * L + offa, L)]
        bb = bb + lbtab_v[pl.ds(t * NT * L + offb, L)]

    for i in range(SEG_PER_W + 1):
        v1 = ba[i + SEG_PER_W] if i < SEG_PER_W else bb[0]
        bnd_s[i] = jnp.where(par == 0, ba[i], v1)

    w = [wb_v[0, pl.ds(k * L, L)] for k in range(KV)]
    bvec = wb_v[1, pl.ds(0, L)]

    zero = jnp.zeros((L,), jnp.float32)
    ninf = jnp.full((L,), -jnp.inf, jnp.float32)

    for j in range(SEG_PER_W):
        for k in range(KV):
            stage_v[j, pl.ds(k * L, L)] = zero
            stage_v[j, pl.ds(D + k * L, L)] = ninf

    r0 = jnp.where(par == 0, ba[0], ba[SEG_PER_W])
    range_end = jnp.where(par == 0, ba[SEG_PER_W], bb[0])
    dbase0 = pl.multiple_of(jnp.minimum(r0 & ~7, N - R), 8)
    nch = jnp.where(r0 < range_end, (range_end - dbase0 + R - 1) // R, 0)

    def issue(c, _):
        cs = pl.multiple_of(jnp.minimum(dbase0 + c * R, N - R), 8)
        pltpu.async_copy(x_hbm.at[pl.ds(cs, R)],
                         xbuf_v.at[pl.ds(pl.multiple_of(c * R, 8), R)], sem)
        return 0

    lax.fori_loop(0, jnp.minimum(nch, NBUF), issue, 0)

    @pl.when(nch > 0)
    def _wait0():
        pltpu.make_async_copy(
            x_hbm.at[pl.ds(0, R)], xbuf_v.at[pl.ds(0, R)], sem).wait()

    def wcond(c):
        return c[0] < range_end

    def wbody(c):
        r, j, ci, dp = c[0], c[1], c[2], c[3]
        maxs = c[4:4 + KV]
        xsp = c[4 + KV:4 + 2 * KV]

        dbase = pl.multiple_of(jnp.minimum(dbase0 + ci * R, N - R), 8)
        seg_end = bnd_s[j + 1]
        chunk_end = dbase + R
        stop = jnp.minimum(seg_end, chunk_end)
        need_next = (stop == chunk_end) & (stop < range_end)

        prow = (ci & (NBUF - 1)) * R + (r - dbase)

        def row(i, c2):
            maxs = c2[:KV]
            xsp = c2[KV:2 * KV]
            dp = c2[2 * KV]
            ri = prow + i
            xs = [xbuf_v[ri, pl.ds(k * L, L)] for k in range(KV)]
            acc = xs[0] * w[0]
            for k in range(1, KV):
                acc = acc + xs[k] * w[k]
            d = jnp.sum(acc)
            sv = 1.0 / (1.0 + jnp.exp(-(dp + bvec)))
            sums = c2[2 * KV + 1:]
            nsums = tuple(sums[k] + sv * xsp[k] for k in range(KV))
            nmaxs = tuple(jnp.maximum(maxs[k], xs[k]) for k in range(KV))
            return nmaxs + tuple(xs) + (d,) + nsums

        st = lax.fori_loop(0, stop - r, row,
                           maxs + xsp + (dp,) + c[4 + 2 * KV:])
        maxs = st[:KV]
        xsp = st[KV:2 * KV]
        dp = st[2 * KV]
        sums = st[2 * KV + 1:]

        def do_flush(op):
            sums, maxs, xsp, dp, j = op
            sv = 1.0 / (1.0 + jnp.exp(-(dp + bvec)))
            for k in range(KV):
                stage_v[j, pl.ds(k * L, L)] = sums[k] + sv * xsp[k]
                stage_v[j, pl.ds(D + k * L, L)] = maxs[k]
            return ((zero,) * KV, (ninf,) * KV, (zero,) * KV,
                    jnp.float32(0.0), j + 1)

        sums, maxs, xsp, dp, j = lax.cond(
            stop == seg_end, do_flush, lambda op: op,
            (tuple(sums), tuple(maxs), tuple(xsp), dp, j))

        @pl.when(need_next & (ci + NBUF < nch))
        def _refill():
            cs = pl.multiple_of(jnp.minimum(dbase0 + (ci + NBUF) * R, N - R), 8)
            pltpu.async_copy(
                x_hbm.at[pl.ds(cs, R)],
                xbuf_v.at[pl.ds(pl.multiple_of((ci & (NBUF - 1)) * R, 8), R)],
                sem)

        @pl.when(need_next)
        def _wait_next():
            pltpu.make_async_copy(
                x_hbm.at[pl.ds(0, R)], xbuf_v.at[pl.ds(0, R)], sem).wait()

        ci = jnp.where(need_next, ci + 1, ci)
        return (stop, j, ci, dp) + tuple(maxs) + tuple(xsp) + tuple(sums)

    init = ((r0, jnp.int32(0), jnp.int32(0), jnp.float32(0.0))
            + (ninf,) * KV + (zero,) * KV + (zero,) * KV)
    lax.while_loop(wcond, wbody, init)

    pltpu.sync_copy(stage_v, out_hbm.at[pl.ds(wid * SEG_PER_W, SEG_PER_W)])


@jax.jit
def kernel(x, batch, W, b):
    batch32 = batch.astype(jnp.int32)
    wb = jnp.concatenate(
        [W.astype(jnp.float32),
         jnp.broadcast_to(b.astype(jnp.float32).reshape(1, 1), (1, D))], axis=0)
    mesh = plsc.VectorSubcoreMesh(core_axis_name="c", subcore_axis_name="s")
    fn = functools.partial(
        pl.kernel,
        out_type=jax.ShapeDtypeStruct((S, 2 * D), jnp.float32),
        mesh=mesh,
        compiler_params=pltpu.CompilerParams(needs_layout_passes=False),
        scratch_types=[
            pltpu.VMEM((SL,), jnp.int32),
            pltpu.VMEM((NT * L,), jnp.int32),
            pltpu.VMEM((NS * NT * L,), jnp.int32),
            pltpu.VMEM((NBUF * R, D), jnp.float32),
            pltpu.VMEM((2, D), jnp.float32),
            pltpu.VMEM((SEG_PER_W, 2 * D), jnp.float32),
            pltpu.SMEM((L,), jnp.int32),
            pltpu.VMEM_SHARED((NS * NT * L,), jnp.int32),
            pltpu.SemaphoreType.DMA,
        ],
    )(_body)
    return fn(x, batch32, wb)


# speedup vs baseline: 1.0197x; 1.0197x over previous
import functools

import jax
import jax.numpy as jnp
from jax import lax
from jax.experimental import pallas as pl
from jax.experimental.pallas import tpu as pltpu
from jax.experimental.pallas import tpu_sc as plsc

N = 100000
D = 128
S = 256
L = 16
NC = 2
NS = 16
NW = NC * NS
SEG_PER_W = S // NW
R = 128
NBUF = 6
KV = D // L
PIECE = 6248
SL = 6288
NT = 17


def _body(x_hbm, batch_hbm, wb_hbm, out_hbm, slice_v, lbv_v, lbtab_v, xbuf_v,
          wb_v, stage_v, bnd_s, lbsh_sh, sem):
    cid = lax.axis_index("c")
    sid = lax.axis_index("s")
    wid = cid * NS + sid

    pltpu.sync_copy(wb_hbm, wb_v)

    bstart = pl.multiple_of(jnp.minimum(sid * PIECE, N - SL), 8)
    pltpu.sync_copy(batch_hbm.at[pl.ds(bstart, SL)], slice_v)
    poff = sid * PIECE - bstart
    plen = jnp.where(sid == NS - 1, N - (NS - 1) * PIECE, PIECE)

    for g in range(NT):
        tg = g * L + lax.iota(jnp.int32, L)
        lo0 = jnp.full((L,), poff, jnp.int32)
        hi0 = jnp.full((L,), poff + plen, jnp.int32)

        def sbody(_, c):
            lo, hi = c
            act = lo < hi
            mid = lax.shift_right_logical(lo + hi, 1)
            vals = plsc.load_gather(slice_v, [jnp.minimum(mid, SL - 1)])
            less = vals < tg
            lo = jnp.where(act & less, mid + 1, lo)
            hi = jnp.where(act & (~less), mid, hi)
            return lo, hi

        lo, _ = lax.fori_loop(0, 13, sbody, (lo0, hi0))
        lbv_v[pl.ds(g * L, L)] = lo - poff

    pltpu.sync_copy(lbv_v, lbsh_sh.at[pl.ds(sid * NT * L, NT * L)])
    plsc.subcore_barrier()
    pltpu.sync_copy(lbsh_sh, lbtab_v)

    g0 = lax.shift_right_logical(wid, 1)
    par = wid & 1
    offa = pl.multiple_of(g0 * L, 16)
    offb = pl.multiple_of(jnp.minimum(g0 + 1, NT - 1) * L, 16)
    ba = lbtab_v[pl.ds(offa, L)]
    bb = lbtab_v[pl.ds(offb, L)]
    for t in range(1, NS):
        ba = ba + lbtab_v[pl.ds(t * NT * L + offa, L)]
        bb = bb + lbtab_v[pl.ds(t * NT * L + offb, L)]

    for i in range(SEG_PER_W + 1):
        v1 = ba[i + SEG_PER_W] if i < SEG_PER_W else bb[0]
        bnd_s[i] = jnp.where(par == 0, ba[i], v1)

    w = [wb_v[0, pl.ds(k * L, L)] for k in range(KV)]
    bvec = wb_v[1, pl.ds(0, L)]

    zero = jnp.zeros((L,), jnp.float32)
    ninf = jnp.full((L,), -jnp.inf, jnp.float32)

    for j in range(SEG_PER_W):
        for k in range(KV):
            stage_v[j, pl.ds(k * L, L)] = zero
            stage_v[j, pl.ds(D + k * L, L)] = ninf

    r0 = jnp.where(par == 0, ba[0], ba[SEG_PER_W])
    range_end = jnp.where(par == 0, ba[SEG_PER_W], bb[0])
    dbase0 = pl.multiple_of(jnp.minimum(r0 & ~7, N - R), 8)
    nch = jnp.where(r0 < range_end, (range_end - dbase0 + R - 1) // R, 0)

    def issue(c, _):
        cs = pl.multiple_of(jnp.minimum(dbase0 + c * R, N - R), 8)
        pltpu.async_copy(x_hbm.at[pl.ds(cs, R)],
                         xbuf_v.at[pl.ds(pl.multiple_of(c * R, 8), R)], sem)
        return 0

    lax.fori_loop(0, jnp.minimum(nch, NBUF), issue, 0)

    @pl.when(nch > 0)
    def _wait0():
        pltpu.make_async_copy(
            x_hbm.at[pl.ds(0, R)], xbuf_v.at[pl.ds(0, R)], sem).wait()

    def wcond(c):
        return c[0] < range_end

    def wbody(c):
        r, j, ci, dp = c[0], c[1], c[2], c[3]
        maxs = c[4:4 + KV]
        xsp = c[4 + KV:4 + 2 * KV]

        dbase = pl.multiple_of(jnp.minimum(dbase0 + ci * R, N - R), 8)
        seg_end = bnd_s[j + 1]
        chunk_end = dbase + R
        stop = jnp.minimum(seg_end, chunk_end)
        need_next = (stop == chunk_end) & (stop < range_end)

        prow = lax.rem(ci, NBUF) * R + (r - dbase)

        def row(i, c2):
            maxs = c2[:KV]
            xsp = c2[KV:2 * KV]
            dp = c2[2 * KV]
            ri = prow + i
            xs = [xbuf_v[ri, pl.ds(k * L, L)] for k in range(KV)]
            acc = xs[0] * w[0]
            for k in range(1, KV):
                acc = acc + xs[k] * w[k]
            d = jnp.sum(acc)
            sv = 1.0 / (1.0 + jnp.exp(-(dp + bvec)))
            sums = c2[2 * KV + 1:]
            nsums = tuple(sums[k] + sv * xsp[k] for k in range(KV))
            nmaxs = tuple(jnp.maximum(maxs[k], xs[k]) for k in range(KV))
            return nmaxs + tuple(xs) + (d,) + nsums

        st = lax.fori_loop(0, stop - r, row,
                           maxs + xsp + (dp,) + c[4 + 2 * KV:])
        maxs = st[:KV]
        xsp = st[KV:2 * KV]
        dp = st[2 * KV]
        sums = st[2 * KV + 1:]

        def do_flush(op):
            sums, maxs, xsp, dp, j = op
            sv = 1.0 / (1.0 + jnp.exp(-(dp + bvec)))
            for k in range(KV):
                stage_v[j, pl.ds(k * L, L)] = sums[k] + sv * xsp[k]
                stage_v[j, pl.ds(D + k * L, L)] = maxs[k]
            return ((zero,) * KV, (ninf,) * KV, (zero,) * KV,
                    jnp.float32(0.0), j + 1)

        sums, maxs, xsp, dp, j = lax.cond(
            stop == seg_end, do_flush, lambda op: op,
            (tuple(sums), tuple(maxs), tuple(xsp), dp, j))

        @pl.when(need_next & (ci + NBUF < nch))
        def _refill():
            cs = pl.multiple_of(jnp.minimum(dbase0 + (ci + NBUF) * R, N - R), 8)
            pltpu.async_copy(
                x_hbm.at[pl.ds(cs, R)],
                xbuf_v.at[pl.ds(pl.multiple_of(lax.rem(ci, NBUF) * R, 8), R)],
                sem)

        @pl.when(need_next)
        def _wait_next():
            pltpu.make_async_copy(
                x_hbm.at[pl.ds(0, R)], xbuf_v.at[pl.ds(0, R)], sem).wait()

        ci = jnp.where(need_next, ci + 1, ci)
        return (stop, j, ci, dp) + tuple(maxs) + tuple(xsp) + tuple(sums)

    init = ((r0, jnp.int32(0), jnp.int32(0), jnp.float32(0.0))
            + (ninf,) * KV + (zero,) * KV + (zero,) * KV)
    lax.while_loop(wcond, wbody, init)

    pltpu.sync_copy(stage_v, out_hbm.at[pl.ds(wid * SEG_PER_W, SEG_PER_W)])


@jax.jit
def kernel(x, batch, W, b):
    batch32 = batch.astype(jnp.int32)
    wb = jnp.concatenate(
        [W.astype(jnp.float32),
         jnp.broadcast_to(b.astype(jnp.float32).reshape(1, 1), (1, D))], axis=0)
    mesh = plsc.VectorSubcoreMesh(core_axis_name="c", subcore_axis_name="s")
    fn = functools.partial(
        pl.kernel,
        out_type=jax.ShapeDtypeStruct((S, 2 * D), jnp.float32),
        mesh=mesh,
        compiler_params=pltpu.CompilerParams(needs_layout_passes=False),
        scratch_types=[
            pltpu.VMEM((SL,), jnp.int32),
            pltpu.VMEM((NT * L,), jnp.int32),
            pltpu.VMEM((NS * NT * L,), jnp.int32),
            pltpu.VMEM((NBUF * R, D), jnp.float32),
            pltpu.VMEM((2, D), jnp.float32),
            pltpu.VMEM((SEG_PER_W, 2 * D), jnp.float32),
            pltpu.SMEM((L,), jnp.int32),
            pltpu.VMEM_SHARED((NS * NT * L,), jnp.int32),
            pltpu.SemaphoreType.DMA,
        ],
    )(_body)
    return fn(x, batch32, wb)
